# baseline (speedup 1.0000x reference)
import functools

import jax
import jax.numpy as jnp
from jax import lax
from jax.experimental import pallas as pl
from jax.experimental.pallas import tpu as pltpu
from jax.experimental.pallas import tpu_sc as plsc

DIM = 64
NW = 32
SUB = 128


def _dequant_gather(qtab, aux, idx2, n1, ncols):
    n_per_w = n1 // NW
    nsub = n_per_w // SUB
    mesh = plsc.VectorSubcoreMesh(core_axis_name="c", subcore_axis_name="s")

    @functools.partial(
        pl.kernel,
        mesh=mesh,
        out_type=jax.ShapeDtypeStruct((ncols, DIM, n1), jnp.float32),
        compiler_params=pltpu.CompilerParams(
            needs_layout_passes=False, use_tc_tiling_on_sc=False),
        scratch_types=[
            pltpu.VMEM((nsub, SUB), jnp.int32),
            pltpu.VMEM((n_per_w, 16), jnp.int32),
            pltpu.VMEM((n_per_w, 16), jnp.float32),
            pltpu.VMEM((DIM, SUB * 4), jnp.float32),
            pltpu.SemaphoreType.DMA,
        ],
    )
    def body(qtab_ref, aux_ref, idx_ref, out_ref,
             idx_v, q_v, a_v, out_v, sem):
        wid = lax.axis_index("s") * 2 + lax.axis_index("c")
        lanes = lax.iota(jnp.int32, 16)
        scol = lanes >> 3
        zcol = scol + 2
        ccols = [lanes * 4 + k for k in range(4)]

        for b in range(ncols):
            ib = b * (n1 // SUB) + wid * nsub
            pltpu.sync_copy(idx_ref.at[pl.ds(ib, nsub)], idx_v)
            copies = []
            for j in range(nsub):
                copies.append(pltpu.async_copy(
                    qtab_ref.at[idx_v.at[j]],
                    q_v.at[pl.ds(j * SUB, SUB)], sem))
                copies.append(pltpu.async_copy(
                    aux_ref.at[idx_v.at[j]],
                    a_v.at[pl.ds(j * SUB, SUB)], sem))
            for cp in copies:
                cp.wait()

            def row_body(nl, carry):
                rsp = jnp.full((16,), nl, jnp.int32)
                qw = plsc.load_gather(q_v, [rsp, lanes])
                sv = plsc.load_gather(a_v, [rsp, scol])
                zv = plsc.load_gather(a_v, [rsp, zcol])
                nsp = jnp.full((16,), nl, jnp.int32)
                for k in range(4):
                    pk = (qw << (24 - 8 * k)) >> 24 if k < 3 else qw >> 24
                    res = (pk.astype(jnp.float32) - zv) * sv
                    plsc.store_scatter(out_v, [ccols[k], nsp], res)
                return carry

            lax.fori_loop(0, n_per_w, row_body, 0)
            pltpu.sync_copy(
                out_v, out_ref.at[b, :, pl.ds(wid * n_per_w, n_per_w)])

    return body(qtab, aux, idx2)


def kernel(packed_weight_qvals, weight_scales, weight_zeros, x):
    V, D = packed_weight_qvals.shape
    n1, ncols = x.shape
    qtab = lax.bitcast_convert_type(
        packed_weight_qvals.reshape(V, 16, 4), jnp.int32)
    aux = jnp.concatenate(
        [weight_scales, weight_zeros.astype(jnp.float32),
         jnp.zeros((V, 12), jnp.float32)], axis=1)
    idx2 = x.T.astype(jnp.int32).reshape(n1 * ncols // SUB, SUB)
    out = _dequant_gather(qtab, aux, idx2, n1, ncols)
    return jnp.transpose(out, (2, 0, 1))


# speedup vs baseline: 1.0348x; 1.0348x over previous
import functools

import jax
import jax.numpy as jnp
from jax import lax
from jax.experimental import pallas as pl
from jax.experimental.pallas import tpu as pltpu
from jax.experimental.pallas import tpu_sc as plsc

DIM = 64
NW = 32
SUB = 128


def _dequant_gather(qtab, aux, idx2, n1, ncols):
    n_per_w = n1 // NW
    nsub = n_per_w // SUB
    mesh = plsc.VectorSubcoreMesh(core_axis_name="c", subcore_axis_name="s")

    @functools.partial(
        pl.kernel,
        mesh=mesh,
        out_type=jax.ShapeDtypeStruct((ncols, DIM, n1), jnp.float32),
        compiler_params=pltpu.CompilerParams(
            needs_layout_passes=False, use_tc_tiling_on_sc=False),
        scratch_types=[
            pltpu.VMEM((nsub, SUB), jnp.int32),
            pltpu.VMEM((n_per_w, 16), jnp.int32),
            pltpu.VMEM((n_per_w, 16), jnp.float32),
            pltpu.VMEM((DIM, SUB * 4 + 1), jnp.float32),
            pltpu.SemaphoreType.DMA,
        ],
    )
    def body(qtab_ref, aux_ref, idx_ref, out_ref,
             idx_v, q_v, a_v, out_v, sem):
        wid = lax.axis_index("s") * 2 + lax.axis_index("c")
        lanes = lax.iota(jnp.int32, 16)
        scol = lanes >> 3
        zcol = scol + 2
        ccols = [lanes * 4 + k for k in range(4)]

        for b in range(ncols):
            ib = b * (n1 // SUB) + wid * nsub
            pltpu.sync_copy(idx_ref.at[pl.ds(ib, nsub)], idx_v)
            copies = []
            for j in range(nsub):
                copies.append(pltpu.async_copy(
                    qtab_ref.at[idx_v.at[j]],
                    q_v.at[pl.ds(j * SUB, SUB)], sem))
                copies.append(pltpu.async_copy(
                    aux_ref.at[idx_v.at[j]],
                    a_v.at[pl.ds(j * SUB, SUB)], sem))
            for cp in copies:
                cp.wait()

            def row_body(nl, carry):
                rsp = jnp.full((16,), nl, jnp.int32)
                qw = plsc.load_gather(q_v, [rsp, lanes])
                sv = plsc.load_gather(a_v, [rsp, scol])
                zv = plsc.load_gather(a_v, [rsp, zcol])
                nsp = jnp.full((16,), nl, jnp.int32)
                for k in range(4):
                    pk = (qw << (24 - 8 * k)) >> 24 if k < 3 else qw >> 24
                    res = (pk.astype(jnp.float32) - zv) * sv
                    plsc.store_scatter(out_v, [ccols[k], nsp], res)
                return carry

            lax.fori_loop(0, n_per_w, row_body, 0)
            pltpu.sync_copy(
                out_v.at[:, pl.ds(0, n_per_w)],
                out_ref.at[b, :, pl.ds(wid * n_per_w, n_per_w)])

    return body(qtab, aux, idx2)


def kernel(packed_weight_qvals, weight_scales, weight_zeros, x):
    V, D = packed_weight_qvals.shape
    n1, ncols = x.shape
    qtab = lax.bitcast_convert_type(
        packed_weight_qvals.reshape(V, 16, 4), jnp.int32)
    aux = jnp.concatenate(
        [weight_scales, weight_zeros.astype(jnp.float32),
         jnp.zeros((V, 12), jnp.float32)], axis=1)
    idx2 = x.T.astype(jnp.int32).reshape(n1 * ncols // SUB, SUB)
    out = _dequant_gather(qtab, aux, idx2, n1, ncols)
    return jnp.transpose(out, (2, 0, 1))


# speedup vs baseline: 1.2731x; 1.2304x over previous
import functools

import jax
import jax.numpy as jnp
from jax import lax
from jax.experimental import pallas as pl
from jax.experimental.pallas import tpu as pltpu
from jax.experimental.pallas import tpu_sc as plsc

DIM = 64
NW = 32
SUB = 128


def _dequant_gather(ctab, idx2, n_flat):
    rows_per_w = n_flat // NW
    chunk = 1280
    nchunks = rows_per_w // chunk
    nsub = chunk // SUB
    mesh = plsc.VectorSubcoreMesh(core_axis_name="c", subcore_axis_name="s")

    @functools.partial(
        pl.kernel,
        mesh=mesh,
        out_type=jax.ShapeDtypeStruct((n_flat // 2, 2 * DIM), jnp.float32),
        compiler_params=pltpu.CompilerParams(
            needs_layout_passes=False, use_tc_tiling_on_sc=False),
        scratch_types=[
            pltpu.VMEM((nsub, SUB), jnp.int32),
            pltpu.VMEM((chunk, 32), jnp.int32),
            pltpu.VMEM((chunk // 2, 2 * DIM), jnp.float32),
            pltpu.SemaphoreType.DMA,
        ],
    )
    def body(ctab_ref, idx_ref, out_ref, idx_v, c_v, out_v, sem):
        wid = lax.axis_index("s") * 2 + lax.axis_index("c")
        lanes = lax.iota(jnp.int32, 16)
        scol = (lanes >> 3) + 16
        zcol = scol + 2
        ccols = [lanes * 4 + k for k in range(4)]

        for c in range(nchunks):
            base = wid * rows_per_w + c * chunk
            ib = wid * (rows_per_w // SUB) + c * nsub
            pltpu.sync_copy(idx_ref.at[pl.ds(ib, nsub)], idx_v)
            copies = []
            for j in range(nsub):
                copies.append(pltpu.async_copy(
                    ctab_ref.at[idx_v.at[j]],
                    c_v.at[pl.ds(j * SUB, SUB)], sem))
            for cp in copies:
                cp.wait()

            def row_body(r, carry):
                rsp = jnp.full((16,), r, jnp.int32)
                qw = plsc.load_gather(c_v, [rsp, lanes])
                sv = plsc.bitcast(
                    plsc.load_gather(c_v, [rsp, scol]), jnp.float32)
                zv = plsc.bitcast(
                    plsc.load_gather(c_v, [rsp, zcol]), jnp.float32)
                orow = jnp.full((16,), r >> 1, jnp.int32)
                obase = jnp.full((16,), (r & 1) << 6, jnp.int32)
                for k in range(4):
                    pk = (qw << (24 - 8 * k)) >> 24 if k < 3 else qw >> 24
                    res = (pk.astype(jnp.float32) - zv) * sv
                    plsc.store_scatter(out_v, [orow, obase + ccols[k]], res)
                return carry

            lax.fori_loop(0, chunk, row_body, 0)
            pltpu.sync_copy(out_v, out_ref.at[pl.ds(base >> 1, chunk // 2)])

    return body(ctab, idx2)


def kernel(packed_weight_qvals, weight_scales, weight_zeros, x):
    V, D = packed_weight_qvals.shape
    qtab = lax.bitcast_convert_type(
        packed_weight_qvals.reshape(V, 16, 4), jnp.int32)
    sbits = lax.bitcast_convert_type(weight_scales, jnp.int32)
    zbits = lax.bitcast_convert_type(
        weight_zeros.astype(jnp.float32), jnp.int32)
    ctab = jnp.concatenate(
        [qtab, sbits, zbits, jnp.zeros((V, 12), jnp.int32)], axis=1)
    flat = x.reshape(-1).astype(jnp.int32)
    n_flat = flat.shape[0]
    idx2 = flat.reshape(n_flat // SUB, SUB)
    out = _dequant_gather(ctab, idx2, n_flat)
    return out.reshape(*x.shape, D)


# speedup vs baseline: 1.2990x; 1.0203x over previous
import functools

import jax
import jax.numpy as jnp
from jax import lax
from jax.experimental import pallas as pl
from jax.experimental.pallas import tpu as pltpu
from jax.experimental.pallas import tpu_sc as plsc

DIM = 64
NW = 32
SUB = 128


def _dequant_gather(ctab, idx2, n_flat):
    rows_per_w = n_flat // NW
    chunk = 640
    nchunks = rows_per_w // chunk
    nsub = chunk // SUB
    mesh = plsc.VectorSubcoreMesh(core_axis_name="c", subcore_axis_name="s")

    @functools.partial(
        pl.kernel,
        mesh=mesh,
        out_type=jax.ShapeDtypeStruct((n_flat // 2, 2 * DIM), jnp.float32),
        compiler_params=pltpu.CompilerParams(
            needs_layout_passes=False, use_tc_tiling_on_sc=False),
        scratch_types=[
            pltpu.VMEM((2, nsub, SUB), jnp.int32),
            pltpu.VMEM((2, chunk, 32), jnp.int32),
            pltpu.VMEM((2, chunk // 2, 2 * DIM), jnp.float32),
            pltpu.SemaphoreType.DMA,
            pltpu.SemaphoreType.DMA,
            pltpu.SemaphoreType.DMA,
            pltpu.SemaphoreType.DMA,
        ],
    )
    def body(ctab_ref, idx_ref, out_ref, idx_v, c_v, out_v,
             sem_g0, sem_g1, sem_w0, sem_w1):
        wid = lax.axis_index("s") * 2 + lax.axis_index("c")
        lanes = lax.iota(jnp.int32, 16)
        scol = (lanes >> 3) + 16
        zcol = scol + 2
        ccols = [lanes * 4 + k for k in range(4)]
        sem_g = [sem_g0, sem_g1]
        sem_w = [sem_w0, sem_w1]

        def fire(c, buf):
            ib = wid * (rows_per_w // SUB) + c * nsub
            pltpu.sync_copy(idx_ref.at[pl.ds(ib, nsub)], idx_v.at[buf])
            return [pltpu.async_copy(
                        ctab_ref.at[idx_v.at[buf].at[j]],
                        c_v.at[buf].at[pl.ds(j * SUB, SUB)], sem_g[buf])
                    for j in range(nsub)]

        pend = {0: fire(0, 0)}
        wb = {}
        for c in range(nchunks):
            buf = c & 1
            if c + 1 < nchunks:
                pend[c + 1] = fire(c + 1, (c + 1) & 1)
            for cp in pend.pop(c):
                cp.wait()
            if c >= 2:
                wb.pop(c - 2).wait()

            def row_body(r, carry):
                rsp = jnp.full((16,), r, jnp.int32)
                qw = plsc.load_gather(c_v.at[buf], [rsp, lanes])
                sv = plsc.bitcast(
                    plsc.load_gather(c_v.at[buf], [rsp, scol]), jnp.float32)
                zv = plsc.bitcast(
                    plsc.load_gather(c_v.at[buf], [rsp, zcol]), jnp.float32)
                orow = jnp.full((16,), r >> 1, jnp.int32)
                obase = jnp.full((16,), (r & 1) << 6, jnp.int32)
                for k in range(4):
                    pk = (qw << (24 - 8 * k)) >> 24 if k < 3 else qw >> 24
                    res = (pk.astype(jnp.float32) - zv) * sv
                    plsc.store_scatter(
                        out_v.at[buf], [orow, obase + ccols[k]], res)
                return carry

            lax.fori_loop(0, chunk, row_body, 0)
            base = wid * rows_per_w + c * chunk
            wb[c] = pltpu.async_copy(
                out_v.at[buf],
                out_ref.at[pl.ds(base >> 1, chunk // 2)], sem_w[buf])
        for c in sorted(wb):
            wb.pop(c).wait()

    return body(ctab, idx2)


def kernel(packed_weight_qvals, weight_scales, weight_zeros, x):
    V, D = packed_weight_qvals.shape
    qtab = lax.bitcast_convert_type(
        packed_weight_qvals.reshape(V, 16, 4), jnp.int32)
    sbits = lax.bitcast_convert_type(weight_scales, jnp.int32)
    zbits = lax.bitcast_convert_type(
        weight_zeros.astype(jnp.float32), jnp.int32)
    ctab = jnp.concatenate(
        [qtab, sbits, zbits, jnp.zeros((V, 12), jnp.int32)], axis=1)
    flat = x.reshape(-1).astype(jnp.int32)
    n_flat = flat.shape[0]
    idx2 = flat.reshape(n_flat // SUB, SUB)
    out = _dequant_gather(ctab, idx2, n_flat)
    return out.reshape(*x.shape, D)
